# Initial kernel scaffold; baseline (speedup 1.0000x reference)
import jax
import jax.numpy as jnp
from jax.experimental import pallas as pl


def kernel(note_tokens, note_durs, note_types, emb_weight, type_emb_weight, dur_w, dur_b):
    raise NotImplementedError("write your pallas kernel here")


# speedup vs baseline: 4.0409x; 4.0409x over previous
import functools
import math

import jax
import jax.numpy as jnp
from jax import lax
from jax.experimental import pallas as pl
from jax.experimental.pallas import tpu as pltpu
from jax.experimental.pallas import tpu_sc as plsc

H = 64
SCALE = float(math.sqrt(H))
NW = 32
CH = 512
GB = 128
NSUB = CH // GB


def _make_encoder(N):
    per_w = N // NW
    chunks = per_w // CH
    mesh = plsc.VectorSubcoreMesh(core_axis_name="c", subcore_axis_name="s")

    @functools.partial(
        pl.kernel,
        mesh=mesh,
        compiler_params=pltpu.CompilerParams(use_tc_tiling_on_sc=False),
        out_type=jax.ShapeDtypeStruct((N, H), jnp.float32),
        scratch_types=[
            pltpu.VMEM((NSUB, GB), jnp.int32),
            pltpu.VMEM((CH, H), jnp.float32),
            pltpu.VMEM((CH,), jnp.int32),
            pltpu.VMEM((CH,), jnp.float32),
            pltpu.VMEM((5, H), jnp.float32),
            pltpu.VMEM((5 * H,), jnp.float32),
            pltpu.VMEM((H,), jnp.float32),
            pltpu.VMEM((H,), jnp.float32),
            pltpu.SemaphoreType.DMA,
        ],
    )
    def enc(tok_hbm, typ_hbm, dur_hbm, emb_hbm, te_hbm, dw_hbm, db_hbm,
            out_hbm, idx_v, rows_v, typ_v, dur_v, te_v, t2_v, dw_v, db_v,
            gsem):
        wid = lax.axis_index("s") * 2 + lax.axis_index("c")
        base = wid * per_w

        pltpu.sync_copy(dw_hbm, dw_v)
        pltpu.sync_copy(db_hbm, db_v)
        pltpu.sync_copy(te_hbm, te_v)
        for r in range(5):
            for j in range(4):
                sl = pl.ds(j * 16, 16)
                t2_v[pl.ds(r * H + j * 16, 16)] = te_v[r, sl] * SCALE + db_v[sl]

        dwv = [dw_v[pl.ds(j * 16, 16)] for j in range(4)]

        def chunk_body(g, carry):
            row0 = base + g * CH
            r128 = wid * (per_w // GB) + g * NSUB
            pltpu.sync_copy(tok_hbm.at[pl.ds(r128, NSUB)], idx_v)
            pltpu.sync_copy(typ_hbm.at[pl.ds(row0, CH)], typ_v)
            pltpu.sync_copy(dur_hbm.at[pl.ds(row0, CH)], dur_v)
            copies = [
                pltpu.async_copy(emb_hbm.at[idx_v.at[jb]],
                                 rows_v.at[pl.ds(jb * GB, GB)], gsem)
                for jb in range(NSUB)
            ]
            for cp in copies:
                cp.wait()

            def grp_body(g, c2):
                dur16 = dur_v[pl.ds(g * 16, 16)]
                typ16 = typ_v[pl.ds(g * 16, 16)]
                for k in range(16):
                    i = g * 16 + k
                    d16 = jnp.full((16,), dur16[k], dtype=jnp.float32)
                    toff = typ16[k] * H
                    for j in range(4):
                        t = t2_v[pl.ds(toff + j * 16, 16)]
                        sl = pl.ds(j * 16, 16)
                        e = rows_v[i, sl]
                        rows_v[i, sl] = e * SCALE + (d16 * dwv[j] + t)
                return c2

            lax.fori_loop(0, CH // 16, grp_body, 0)
            pltpu.sync_copy(rows_v, out_hbm.at[pl.ds(row0, CH)])
            return carry

        lax.fori_loop(0, chunks, chunk_body, 0)

    return enc


def kernel(note_tokens, note_durs, note_types, emb_weight, type_emb_weight,
           dur_w, dur_b):
    B, L = note_tokens.shape
    N = B * L
    enc = _make_encoder(N)
    tok = note_tokens.reshape(N // GB, GB).astype(jnp.int32)
    typ = note_types.reshape(N).astype(jnp.int32)
    dur = note_durs.reshape(N)
    out = enc(tok, typ, dur, emb_weight, type_emb_weight, dur_w, dur_b)
    return out.reshape(B, L, H)
